# baseline (speedup 1.0000x reference)
import functools

import jax
import jax.numpy as jnp
from jax import lax
from jax.experimental import pallas as pl
from jax.experimental.pallas import tpu as pltpu
from jax.experimental.pallas import tpu_sc as plsc

B = 128
V = 100000
L = 16
NC = 2
NS = 16
NW = NC * NS
ROWS_PER_W = B // NW
CHUNK = 10000
NCHUNKS = V // CHUNK
CHUNK_VREGS = CHUNK // L


@functools.cache
def _inv_q():
    q = jax.random.exponential(jax.random.key(1), (B, V), dtype=jnp.float32)
    return 1.0 / q


_mesh = plsc.VectorSubcoreMesh(core_axis_name="c", subcore_axis_name="s")


@functools.partial(
    pl.kernel,
    out_type=(
        jax.ShapeDtypeStruct((B, V), jnp.float32),
        jax.ShapeDtypeStruct((B, L), jnp.int32),
    ),
    mesh=_mesh,
    compiler_params=pltpu.CompilerParams(use_tc_tiling_on_sc=False,
                                         needs_layout_passes=False),
    scratch_types=[
        pltpu.VMEM((V,), jnp.float32),
        pltpu.VMEM((CHUNK,), jnp.float32),
        pltpu.VMEM((L,), jnp.int32),
    ],
)
def _sampler_kernel(logits_hbm, invq_hbm, probs_hbm, samp_hbm,
                    row_v, q_v, idx_v):
    wid = lax.axis_index("s") * NC + lax.axis_index("c")
    lane = lax.iota(jnp.int32, L)

    def do_row(rr, _):
        r = wid * ROWS_PER_W + rr
        pltpu.sync_copy(logits_hbm.at[r], row_v)

        def max_body(i, m):
            return jnp.maximum(m, row_v[pl.ds(i * L, L)])
        m16 = lax.fori_loop(0, V // L, max_body,
                            jnp.full((L,), -jnp.inf, jnp.float32), unroll=8)
        m = jnp.max(m16)

        def exp_body(i, s):
            t = jnp.exp(row_v[pl.ds(i * L, L)] - m)
            row_v[pl.ds(i * L, L)] = t
            return s + t
        s16 = lax.fori_loop(0, V // L, exp_body,
                            jnp.zeros((L,), jnp.float32), unroll=4)
        c = jnp.ones((L,), jnp.float32) / jnp.broadcast_to(jnp.sum(s16), (L,))

        rm = jnp.full((L,), -1.0, jnp.float32)
        ri = jnp.zeros((L,), jnp.int32)
        for ch in range(NCHUNKS):
            base = ch * CHUNK
            pltpu.sync_copy(invq_hbm.at[r, pl.ds(base, CHUNK)], q_v)

            def samp_body(i, carry, base=base):
                rm, ri = carry
                p = row_v[pl.ds(base + i * L, L)] * c
                row_v[pl.ds(base + i * L, L)] = p
                rv = p * q_v[pl.ds(i * L, L)]
                upd = rv > rm
                idx = lane + (base + i * L)
                return (jnp.where(upd, rv, rm), jnp.where(upd, idx, ri))
            rm, ri = lax.fori_loop(0, CHUNK_VREGS, samp_body, (rm, ri),
                                   unroll=4)

        big = jnp.max(rm)
        cand = jnp.where(rm == big, ri, jnp.int32(2**31 - 1))
        samp = jnp.min(cand)

        pltpu.sync_copy(row_v, probs_hbm.at[r])
        idx_v[...] = jnp.broadcast_to(samp, (L,)).astype(jnp.int32)
        pltpu.sync_copy(idx_v, samp_hbm.at[r])
        return 0

    lax.fori_loop(0, ROWS_PER_W, do_row, 0)


def kernel(logits):
    logits32 = logits.astype(jnp.float32)
    probs, samp = _sampler_kernel(logits32, _inv_q())
    return (logits32, probs, samp[:, 0])
